# baseline (speedup 1.0000x reference)
import functools

import jax
import jax.numpy as jnp
from jax.experimental import pallas as pl
from jax.experimental.pallas import tpu as pltpu

TOP_K = 2
NUM_EXPERTS = 8
D_MODEL = 1024
TOKENS = 4096
TBLK = 512


def _moe_block(xb_ref, wg_ref, bg_ref, we_ref, be_ref, out_ref, probs_ref):
    xb = xb_ref[...]
    scores = jnp.dot(xb, wg_ref[...], preferred_element_type=jnp.float32)
    scores = scores + bg_ref[...]
    idx = jax.lax.broadcasted_iota(jnp.int32, scores.shape, 1)
    m1 = jnp.max(scores, axis=1, keepdims=True)
    i1 = jnp.min(jnp.where(scores == m1, idx, NUM_EXPERTS), axis=1,
                 keepdims=True)
    masked = jnp.where(idx == i1, -jnp.inf, scores)
    m2 = jnp.max(masked, axis=1, keepdims=True)
    i2 = jnp.min(jnp.where(masked == m2, idx, NUM_EXPERTS), axis=1,
                 keepdims=True)
    e2 = jnp.exp(m2 - m1)
    denom = 1.0 + e2
    p0 = 1.0 / denom
    p1 = e2 / denom
    probs_ref[...] = jnp.concatenate([p0, p1], axis=1)
    gate = jnp.where(idx == i1, p0, 0.0) + jnp.where(idx == i2, p1, 0.0)

    bias = jnp.dot(gate, be_ref[...], preferred_element_type=jnp.float32)
    gate_bf = gate.astype(jnp.bfloat16)
    y = None
    for e in range(NUM_EXPERTS):
        xg = xb * gate_bf[:, e:e + 1]
        t = jnp.dot(xg, we_ref[e], preferred_element_type=jnp.float32)
        y = t if y is None else y + t
    out_ref[...] = y + bias


def kernel(inputs, Wg, bg, We, be):
    n_tb = TOKENS // TBLK
    out, probs = pl.pallas_call(
        _moe_block,
        grid=(n_tb,),
        in_specs=[
            pl.BlockSpec((TBLK, D_MODEL), lambda t: (t, 0)),
            pl.BlockSpec((D_MODEL, NUM_EXPERTS), lambda t: (0, 0)),
            pl.BlockSpec((1, NUM_EXPERTS), lambda t: (0, 0)),
            pl.BlockSpec((NUM_EXPERTS, D_MODEL, D_MODEL), lambda t: (0, 0, 0)),
            pl.BlockSpec((NUM_EXPERTS, D_MODEL), lambda t: (0, 0)),
        ],
        out_specs=[
            pl.BlockSpec((TBLK, D_MODEL), lambda t: (t, 0)),
            pl.BlockSpec((TBLK, TOP_K), lambda t: (t, 0)),
        ],
        out_shape=[
            jax.ShapeDtypeStruct((TOKENS, D_MODEL), jnp.float32),
            jax.ShapeDtypeStruct((TOKENS, TOP_K), jnp.float32),
        ],
    )(inputs.astype(jnp.bfloat16), Wg, bg.reshape(1, NUM_EXPERTS),
      We.astype(jnp.bfloat16), be)
    return (out, probs)


# speedup vs baseline: 1.0337x; 1.0337x over previous
import functools

import jax
import jax.numpy as jnp
from jax.experimental import pallas as pl
from jax.experimental.pallas import tpu as pltpu

TOP_K = 2
NUM_EXPERTS = 8
D_MODEL = 1024
TOKENS = 4096
TBLK = 256


def _moe_block(xb_ref, wg_ref, bg_ref, we_ref, be_ref, out_ref, probs_ref):
    xb = xb_ref[...]
    scores = jnp.dot(xb, wg_ref[...], preferred_element_type=jnp.float32)
    scores = scores + bg_ref[...]
    idx = jax.lax.broadcasted_iota(jnp.int32, scores.shape, 1)
    m1 = jnp.max(scores, axis=1, keepdims=True)
    i1 = jnp.min(jnp.where(scores == m1, idx, NUM_EXPERTS), axis=1,
                 keepdims=True)
    masked = jnp.where(idx == i1, -jnp.inf, scores)
    m2 = jnp.max(masked, axis=1, keepdims=True)
    i2 = jnp.min(jnp.where(masked == m2, idx, NUM_EXPERTS), axis=1,
                 keepdims=True)
    e2 = jnp.exp(m2 - m1)
    denom = 1.0 + e2
    p0 = 1.0 / denom
    p1 = e2 / denom
    probs_ref[...] = jnp.concatenate([p0, p1], axis=1)
    gate = jnp.where(idx == i1, p0, 0.0) + jnp.where(idx == i2, p1, 0.0)

    acc0 = jnp.dot(gate, be_ref[...], preferred_element_type=jnp.float32)
    acc1 = jnp.zeros((TBLK, D_MODEL), jnp.float32)
    accs = [acc0, acc1]
    for e in range(NUM_EXPERTS):
        y = jnp.dot(xb, we_ref[e], preferred_element_type=jnp.float32)
        accs[e % 2] = accs[e % 2] + gate[:, e:e + 1] * y
    out_ref[...] = accs[0] + accs[1]


def kernel(inputs, Wg, bg, We, be):
    n_tb = TOKENS // TBLK
    out, probs = pl.pallas_call(
        _moe_block,
        grid=(n_tb,),
        in_specs=[
            pl.BlockSpec((TBLK, D_MODEL), lambda t: (t, 0)),
            pl.BlockSpec((D_MODEL, NUM_EXPERTS), lambda t: (0, 0)),
            pl.BlockSpec((1, NUM_EXPERTS), lambda t: (0, 0)),
            pl.BlockSpec((NUM_EXPERTS, D_MODEL, D_MODEL), lambda t: (0, 0, 0)),
            pl.BlockSpec((NUM_EXPERTS, D_MODEL), lambda t: (0, 0)),
        ],
        out_specs=[
            pl.BlockSpec((TBLK, D_MODEL), lambda t: (t, 0)),
            pl.BlockSpec((TBLK, TOP_K), lambda t: (t, 0)),
        ],
        out_shape=[
            jax.ShapeDtypeStruct((TOKENS, D_MODEL), jnp.float32),
            jax.ShapeDtypeStruct((TOKENS, TOP_K), jnp.float32),
        ],
    )(inputs.astype(jnp.bfloat16), Wg, bg.reshape(1, NUM_EXPERTS),
      We.astype(jnp.bfloat16), be)
    return (out, probs)


# speedup vs baseline: 1.0659x; 1.0311x over previous
import functools

import jax
import jax.numpy as jnp
from jax.experimental import pallas as pl
from jax.experimental.pallas import tpu as pltpu

TOP_K = 2
NUM_EXPERTS = 8
D_MODEL = 1024
TOKENS = 4096
TBLK = 1024


def _moe_block(xb_ref, wg_ref, bg_ref, we_ref, be_ref, out_ref, probs_ref):
    xb = xb_ref[...]
    scores = jnp.dot(xb, wg_ref[...], preferred_element_type=jnp.float32)
    scores = scores + bg_ref[...]
    idx = jax.lax.broadcasted_iota(jnp.int32, scores.shape, 1)
    m1 = jnp.max(scores, axis=1, keepdims=True)
    i1 = jnp.min(jnp.where(scores == m1, idx, NUM_EXPERTS), axis=1,
                 keepdims=True)
    masked = jnp.where(idx == i1, -jnp.inf, scores)
    m2 = jnp.max(masked, axis=1, keepdims=True)
    i2 = jnp.min(jnp.where(masked == m2, idx, NUM_EXPERTS), axis=1,
                 keepdims=True)
    e2 = jnp.exp(m2 - m1)
    denom = 1.0 + e2
    p0 = 1.0 / denom
    p1 = e2 / denom
    probs_ref[...] = jnp.concatenate([p0, p1], axis=1)
    gate = jnp.where(idx == i1, p0, 0.0) + jnp.where(idx == i2, p1, 0.0)

    acc0 = jnp.dot(gate, be_ref[...], preferred_element_type=jnp.float32)
    acc1 = jnp.zeros((TBLK, D_MODEL), jnp.float32)
    accs = [acc0, acc1]
    for e in range(NUM_EXPERTS):
        y = jnp.dot(xb, we_ref[e], preferred_element_type=jnp.float32)
        accs[e % 2] = accs[e % 2] + gate[:, e:e + 1] * y
    out_ref[...] = accs[0] + accs[1]


def kernel(inputs, Wg, bg, We, be):
    n_tb = TOKENS // TBLK
    out, probs = pl.pallas_call(
        _moe_block,
        grid=(n_tb,),
        in_specs=[
            pl.BlockSpec((TBLK, D_MODEL), lambda t: (t, 0)),
            pl.BlockSpec((D_MODEL, NUM_EXPERTS), lambda t: (0, 0)),
            pl.BlockSpec((1, NUM_EXPERTS), lambda t: (0, 0)),
            pl.BlockSpec((NUM_EXPERTS, D_MODEL, D_MODEL), lambda t: (0, 0, 0)),
            pl.BlockSpec((NUM_EXPERTS, D_MODEL), lambda t: (0, 0)),
        ],
        out_specs=[
            pl.BlockSpec((TBLK, D_MODEL), lambda t: (t, 0)),
            pl.BlockSpec((TBLK, TOP_K), lambda t: (t, 0)),
        ],
        out_shape=[
            jax.ShapeDtypeStruct((TOKENS, D_MODEL), jnp.float32),
            jax.ShapeDtypeStruct((TOKENS, TOP_K), jnp.float32),
        ],
    )(inputs.astype(jnp.bfloat16), Wg, bg.reshape(1, NUM_EXPERTS),
      We.astype(jnp.bfloat16), be)
    return (out, probs)


# speedup vs baseline: 1.0678x; 1.0018x over previous
import jax
import jax.numpy as jnp
from jax.experimental import pallas as pl

TOP_K = 2
NUM_EXPERTS = 8
D_MODEL = 1024
TOKENS = 4096
TBLK = 1024


def _moe_block(xb_ref, wg_ref, bg_ref, we_ref, be_ref, out_ref, probs_ref):
    xb = xb_ref[...]
    scores = jnp.dot(xb, wg_ref[...], preferred_element_type=jnp.float32)
    scores = scores + bg_ref[...]
    idx = jax.lax.broadcasted_iota(jnp.int32, scores.shape, 1)
    m1 = jnp.max(scores, axis=1, keepdims=True)
    i1 = jnp.min(jnp.where(scores == m1, idx, NUM_EXPERTS), axis=1,
                 keepdims=True)
    masked = jnp.where(idx == i1, -jnp.inf, scores)
    m2 = jnp.max(masked, axis=1, keepdims=True)
    i2 = jnp.min(jnp.where(masked == m2, idx, NUM_EXPERTS), axis=1,
                 keepdims=True)
    e2 = jnp.exp(m2 - m1)
    denom = 1.0 + e2
    p0 = 1.0 / denom
    p1 = e2 / denom
    probs_ref[...] = jnp.concatenate([p0, p1], axis=1)
    gate = jnp.where(idx == i1, p0, 0.0) + jnp.where(idx == i2, p1, 0.0)

    acc0 = jnp.dot(gate, be_ref[...], preferred_element_type=jnp.float32)
    acc1 = jnp.zeros((TBLK, D_MODEL), jnp.float32)
    accs = [acc0, acc1]
    for e in range(NUM_EXPERTS):
        y = jnp.dot(xb, we_ref[e], preferred_element_type=jnp.float32)
        accs[e % 2] = accs[e % 2] + gate[:, e:e + 1] * y
    out_ref[...] = accs[0] + accs[1]


def kernel(inputs, Wg, bg, We, be):
    n_tb = TOKENS // TBLK
    out, probs = pl.pallas_call(
        _moe_block,
        grid=(n_tb,),
        in_specs=[
            pl.BlockSpec((TBLK, D_MODEL), lambda t: (t, 0)),
            pl.BlockSpec((D_MODEL, NUM_EXPERTS), lambda t: (0, 0)),
            pl.BlockSpec((1, NUM_EXPERTS), lambda t: (0, 0)),
            pl.BlockSpec((NUM_EXPERTS, D_MODEL, D_MODEL), lambda t: (0, 0, 0)),
            pl.BlockSpec((NUM_EXPERTS, D_MODEL), lambda t: (0, 0)),
        ],
        out_specs=[
            pl.BlockSpec((TBLK, D_MODEL), lambda t: (t, 0)),
            pl.BlockSpec((TBLK, TOP_K), lambda t: (t, 0)),
        ],
        out_shape=[
            jax.ShapeDtypeStruct((TOKENS, D_MODEL), jnp.float32),
            jax.ShapeDtypeStruct((TOKENS, TOP_K), jnp.float32),
        ],
    )(inputs.astype(jnp.bfloat16), Wg, bg.reshape(1, NUM_EXPERTS),
      We.astype(jnp.bfloat16), be)
    return (out, probs)
